# Initial kernel scaffold; baseline (speedup 1.0000x reference)
import jax
import jax.numpy as jnp
from jax.experimental import pallas as pl


def kernel(h, e, edge_index, W_enc, b_enc, W1, b1, g1, be1, W2, b2, g2, be2, Wr1, br1, Wr2, br2, Wr3, br3):
    raise NotImplementedError("write your pallas kernel here")


# speedup vs baseline: 7.3329x; 7.3329x over previous
import functools

import jax
import jax.numpy as jnp
from jax import lax
from jax.experimental import pallas as pl
from jax.experimental.pallas import tpu as pltpu
from jax.experimental.pallas import tpu_sc as plsc

_NCORES = 2
_NSUB = 16
_NW = _NCORES * _NSUB
_LANES = 128


def _cdiv(a, b):
    return (a + b - 1) // b


def _nacc(n):
    return _cdiv(n + 64, 128) * 128


@functools.lru_cache(maxsize=None)
def _deg_kernel(n_nodes, r_rows):
    nacc = _nacc(n_nodes)
    rows_pt = nacc // _NSUB
    mesh = plsc.VectorSubcoreMesh(core_axis_name="c", subcore_axis_name="s")

    @functools.partial(
        pl.kernel,
        out_type=jax.ShapeDtypeStruct((_NCORES, nacc, 16), jnp.float32),
        mesh=mesh,
        scratch_types=[
            pltpu.VMEM((r_rows, _LANES), jnp.int32),
            pltpu.VMEM((_LANES, 16), jnp.float32),
            pltpu.VMEM((_LANES, 16), jnp.float32),
            pltpu.VMEM_SHARED((nacc, 16), jnp.float32),
        ],
    )
    def deg(dst_hbm, out_hbm, dst_v, ones_v, zeros_v, acc):
        c = lax.axis_index("c")
        s = lax.axis_index("s")
        w = c * _NSUB + s

        def fill(i, _):
            ones_v[i, :] = jnp.full((16,), 1.0, jnp.float32)
            zeros_v[i, :] = jnp.zeros((16,), jnp.float32)
            return 0

        lax.fori_loop(0, _LANES, fill, 0)

        z0 = s * rows_pt
        off = 0
        while off < rows_pt:
            nn = min(_LANES, rows_pt - off)
            pltpu.sync_copy(zeros_v.at[pl.ds(0, nn)], acc.at[pl.ds(z0 + off, nn)])
            off += nn
        plsc.subcore_barrier()

        pltpu.sync_copy(dst_hbm.at[w], dst_v)

        def body(j, _):
            pltpu.sync_copy(ones_v, acc.at[dst_v.at[j]], add=True)
            return 0

        lax.fori_loop(0, r_rows, body, 0)
        plsc.subcore_barrier()

        pltpu.sync_copy(
            acc.at[pl.ds(s * rows_pt, rows_pt)],
            out_hbm.at[c, pl.ds(s * rows_pt, rows_pt)],
        )

    return deg


@functools.lru_cache(maxsize=None)
def _agg_kernel(n_nodes, d, r_rows):
    nacc = _nacc(n_nodes)
    rows_pt = nacc // _NSUB
    mesh = plsc.VectorSubcoreMesh(core_axis_name="c", subcore_axis_name="s")

    @functools.partial(
        pl.kernel,
        out_type=jax.ShapeDtypeStruct((_NCORES, nacc, d), jnp.float32),
        mesh=mesh,
        scratch_types=[
            pltpu.VMEM((r_rows, _LANES), jnp.int32),
            pltpu.VMEM((r_rows, _LANES), jnp.int32),
            pltpu.VMEM((_LANES, 128), jnp.float32),
            pltpu.VMEM_SHARED((nacc, 128), jnp.float32),
            pltpu.SemaphoreType.DMA,
        ],
    )
    def agg(src_hbm, dst_hbm, xn_hbm, out_hbm, src_v, dst_v, rows_v, acc, sem):
        c = lax.axis_index("c")
        s = lax.axis_index("s")
        w = c * _NSUB + s

        def zf(t, _):
            rows_v[t // 8, pl.ds((t % 8) * 16, 16)] = jnp.zeros((16,), jnp.float32)
            return 0

        lax.fori_loop(0, _LANES * 8, zf, 0)

        z0 = s * rows_pt
        off = 0
        while off < rows_pt:
            nn = min(_LANES, rows_pt - off)
            pltpu.sync_copy(rows_v.at[pl.ds(0, nn)], acc.at[pl.ds(z0 + off, nn)])
            off += nn
        plsc.subcore_barrier()

        pltpu.sync_copy(src_hbm.at[w], src_v)
        pltpu.sync_copy(dst_hbm.at[w], dst_v)

        def body(j, _):
            pltpu.async_copy(xn_hbm.at[src_v.at[j]], rows_v, sem).wait()
            pltpu.sync_copy(rows_v, acc.at[dst_v.at[j]], add=True)
            return 0

        lax.fori_loop(0, r_rows, body, 0)
        plsc.subcore_barrier()

        pltpu.sync_copy(
            acc.at[pl.ds(s * rows_pt, rows_pt)],
            out_hbm.at[c, pl.ds(s * rows_pt, rows_pt)],
        )

    return agg


def _enc_body(degp_ref, h_ref, we_ref, be_ref, x_ref, xn_ref, norm_ref):
    n = h_ref.shape[0]
    na = degp_ref.shape[0] // 2
    degp = degp_ref[...]
    deg = degp[0:n, 0:1] + degp[na : na + n, 0:1]
    deg = jnp.maximum(deg, 1.0)
    norm = lax.rsqrt(deg)
    x = jnp.dot(h_ref[...], we_ref[...], preferred_element_type=jnp.float32)
    x = x + be_ref[...][None, :]
    x_ref[...] = x
    xn_ref[...] = x * norm
    norm_ref[...] = norm


def _layer_body(x_ref, p_ref, norm_ref, w1_ref, b1_ref, g1_ref, be1_ref,
                w2_ref, b2_ref, g2_ref, be2_ref, xo_ref, xno_ref):
    norm = norm_ref[...]
    n = x_ref.shape[0]
    agg = (p_ref[0, 0:n, :] + p_ref[1, 0:n, :]) * norm
    y = jnp.dot(agg, w1_ref[...], preferred_element_type=jnp.float32)
    y = y + b1_ref[...][None, :]
    mu = jnp.mean(y, axis=0, keepdims=True)
    var = jnp.mean((y - mu) ** 2, axis=0, keepdims=True)
    y = (y - mu) * lax.rsqrt(var + 1e-5) * g1_ref[...][None, :] + be1_ref[...][None, :]
    y = jnp.maximum(y, 0.0)
    y = jnp.dot(y, w2_ref[...], preferred_element_type=jnp.float32)
    y = y + b2_ref[...][None, :]
    mu2 = jnp.mean(y, axis=0, keepdims=True)
    var2 = jnp.mean((y - mu2) ** 2, axis=0, keepdims=True)
    y = (y - mu2) * lax.rsqrt(var2 + 1e-5) * g2_ref[...][None, :] + be2_ref[...][None, :]
    y = jnp.maximum(y, 0.0)
    x = x_ref[...] + y
    xo_ref[...] = x
    xno_ref[...] = x * norm


def _readout_body(x_ref, wr1_ref, br1_ref, wr2_ref, br2_ref, wr3_ref, br3_ref,
                  o_ref):
    hg = jnp.mean(x_ref[...], axis=0, keepdims=True)
    r = jnp.dot(hg, wr1_ref[...], preferred_element_type=jnp.float32)
    r = jnp.maximum(r + br1_ref[...][None, :], 0.0)
    r = jnp.dot(r, wr2_ref[...], preferred_element_type=jnp.float32)
    r = jnp.maximum(r + br2_ref[...][None, :], 0.0)
    r = jnp.dot(r, wr3_ref[...], preferred_element_type=jnp.float32)
    o_ref[...] = r + br3_ref[...][None, :]


def kernel(h, e, edge_index, W_enc, b_enc, W1, b1, g1, be1, W2, b2, g2, be2,
           Wr1, br1, Wr2, br2, Wr3, br3):
    n, d = h.shape
    num_layers = W1.shape[0]
    nc = Wr3.shape[1]
    src = edge_index[0]
    dst = edge_index[1]
    n_edges = src.shape[0]

    r_rows = _cdiv(_cdiv(n_edges, _LANES), _NW)
    e_pad = _NW * r_rows * _LANES
    pad = e_pad - n_edges
    n_dummy = _nacc(n) - n
    if pad:
        ar = jnp.arange(pad, dtype=src.dtype)
        src = jnp.concatenate([src, (ar * 7919) % n])
        dst = jnp.concatenate([dst, n + (ar % n_dummy)])
    src_rows = src.reshape(_NW, r_rows, _LANES)
    dst_rows = dst.reshape(_NW, r_rows, _LANES)

    degp = _deg_kernel(n, r_rows)(dst_rows)

    f32 = jnp.float32
    x, xn, norm = pl.pallas_call(
        _enc_body,
        out_shape=(
            jax.ShapeDtypeStruct((n, d), f32),
            jax.ShapeDtypeStruct((n, d), f32),
            jax.ShapeDtypeStruct((n, 1), f32),
        ),
    )(degp.reshape(_NCORES * _nacc(n), 16), h, W_enc, b_enc)

    layer_call = pl.pallas_call(
        _layer_body,
        out_shape=(
            jax.ShapeDtypeStruct((n, d), f32),
            jax.ShapeDtypeStruct((n, d), f32),
        ),
    )
    agg = _agg_kernel(n, d, r_rows)
    for i in range(num_layers):
        parts = agg(src_rows, dst_rows, xn)
        x, xn = layer_call(x, parts, norm, W1[i], b1[i], g1[i], be1[i],
                           W2[i], b2[i], g2[i], be2[i])

    out = pl.pallas_call(
        _readout_body,
        out_shape=jax.ShapeDtypeStruct((1, nc), f32),
    )(x, Wr1, br1, Wr2, br2, Wr3, br3)
    return out
